# Initial kernel scaffold; baseline (speedup 1.0000x reference)
import functools

import jax
import jax.numpy as jnp
from jax import lax
from jax.experimental import pallas as pl
from jax.experimental.pallas import tpu as pltpu

N_TOKENS = 2048
D_IN = 768
D_OUT = 768
N_EXPERTS = 64

BLK = 128
NB = N_TOKENS // BLK
NSTEPS = NB + N_EXPERTS - 1

_INTERPRET = False


def _gmm_body(blk_ref, exp_ref, lo_ref, hi_ref, xs_ref, w_ref, b_ref, o_ref):
    s = pl.program_id(0)
    lo = lo_ref[s]
    hi = hi_ref[s]
    blk = blk_ref[s]

    @pl.when(hi > lo)
    def _():
        rows = blk * BLK + lax.broadcasted_iota(jnp.int32, (BLK, 1), 0)
        mask = (rows >= lo) & (rows < hi)
        y = lax.dot_general(
            xs_ref[...], w_ref[0],
            (((1,), (1,)), ((), ())),
            preferred_element_type=jnp.float32,
        )
        y = jnp.maximum(y + b_ref[...], 0.0)
        o_ref[...] = jnp.where(mask, y, o_ref[...])


def _grouped_matmul(x_sorted, W, b, blocks, experts, lo, hi):
    grid_spec = pltpu.PrefetchScalarGridSpec(
        num_scalar_prefetch=4,
        grid=(NSTEPS,),
        in_specs=[
            pl.BlockSpec((BLK, D_IN), lambda s, bs, es, ls, hs: (bs[s], 0)),
            pl.BlockSpec((1, D_OUT, D_IN), lambda s, bs, es, ls, hs: (es[s], 0, 0)),
            pl.BlockSpec((1, D_OUT), lambda s, bs, es, ls, hs: (es[s], 0)),
        ],
        out_specs=pl.BlockSpec((BLK, D_OUT), lambda s, bs, es, ls, hs: (bs[s], 0)),
    )
    return pl.pallas_call(
        _gmm_body,
        grid_spec=grid_spec,
        out_shape=jax.ShapeDtypeStruct((N_TOKENS, D_OUT), jnp.float32),
        compiler_params=pltpu.CompilerParams(
            dimension_semantics=("arbitrary",),
        ),
        interpret=_INTERPRET,
    )(blocks, experts, lo, hi, x_sorted, W, b)


def _schedule(offs):
    bounds = jnp.arange(1, NB, dtype=jnp.int32) * BLK
    starts = jnp.sort(jnp.concatenate([offs, bounds]))
    ends = jnp.concatenate([starts[1:], jnp.array([N_TOKENS], jnp.int32)])
    experts = jnp.clip(
        jnp.searchsorted(offs, starts, side="right").astype(jnp.int32) - 1,
        0, N_EXPERTS - 1)
    blocks = jnp.clip(starts // BLK, 0, NB - 1).astype(jnp.int32)
    return blocks, experts, starts.astype(jnp.int32), ends.astype(jnp.int32)


def kernel(x, idxs, W, b):
    idxs = idxs.astype(jnp.int32)
    counts = jnp.bincount(idxs, length=N_EXPERTS).astype(jnp.int32)
    offs = (jnp.cumsum(counts) - counts).astype(jnp.int32)
    order = jnp.argsort(idxs)
    x_sorted = jnp.take(x, order, axis=0)
    blocks, experts, lo, hi = _schedule(offs)
    y_sorted = _grouped_matmul(x_sorted, W, b, blocks, experts, lo, hi)
    inv = jnp.zeros((N_TOKENS,), jnp.int32).at[order].set(
        jnp.arange(N_TOKENS, dtype=jnp.int32))
    return jnp.take(y_sorted, inv, axis=0)


# speedup vs baseline: 3.4925x; 3.4925x over previous
import functools

import jax
import jax.numpy as jnp
from jax import lax
from jax.experimental import pallas as pl
from jax.experimental.pallas import tpu as pltpu

N_TOKENS = 2048
D_IN = 768
D_OUT = 768
N_EXPERTS = 64

BLK = 128
NB = N_TOKENS // BLK
NSTEPS = NB + N_EXPERTS - 1

_INTERPRET = False


def _gmm_body(blk_ref, exp_ref, lo_ref, hi_ref, xs_ref, w_ref, b_ref, o_ref):
    s = pl.program_id(0)
    lo = lo_ref[s]
    hi = hi_ref[s]
    blk = blk_ref[s]

    @pl.when(hi > lo)
    def _():
        rows = blk * BLK + lax.broadcasted_iota(jnp.int32, (BLK, 1), 0)
        mask = (rows >= lo) & (rows < hi)
        y = lax.dot_general(
            xs_ref[...], w_ref[0],
            (((1,), (1,)), ((), ())),
            preferred_element_type=jnp.float32,
        )
        y = jnp.maximum(y + b_ref[0], 0.0)
        o_ref[...] = jnp.where(mask, y, o_ref[...])


def _grouped_matmul(x_sorted, W, b, blocks, experts, lo, hi):
    grid_spec = pltpu.PrefetchScalarGridSpec(
        num_scalar_prefetch=4,
        grid=(NSTEPS,),
        in_specs=[
            pl.BlockSpec((BLK, D_IN), lambda s, bs, es, ls, hs: (bs[s], 0)),
            pl.BlockSpec((1, D_OUT, D_IN), lambda s, bs, es, ls, hs: (es[s], 0, 0)),
            pl.BlockSpec((1, 1, D_OUT), lambda s, bs, es, ls, hs: (es[s], 0, 0)),
        ],
        out_specs=pl.BlockSpec((BLK, D_OUT), lambda s, bs, es, ls, hs: (bs[s], 0)),
    )
    return pl.pallas_call(
        _gmm_body,
        grid_spec=grid_spec,
        out_shape=jax.ShapeDtypeStruct((N_TOKENS, D_OUT), jnp.float32),
        compiler_params=pltpu.CompilerParams(
            dimension_semantics=("arbitrary",),
        ),
        interpret=_INTERPRET,
    )(blocks, experts, lo, hi, x_sorted, W, b.reshape(N_EXPERTS, 1, D_OUT))


def _schedule(offs):
    bounds = jnp.arange(1, NB, dtype=jnp.int32) * BLK
    starts = jnp.sort(jnp.concatenate([offs, bounds]))
    ends = jnp.concatenate([starts[1:], jnp.array([N_TOKENS], jnp.int32)])
    experts = jnp.clip(
        jnp.searchsorted(offs, starts, side="right").astype(jnp.int32) - 1,
        0, N_EXPERTS - 1)
    blocks = jnp.clip(starts // BLK, 0, NB - 1).astype(jnp.int32)
    return blocks, experts, starts.astype(jnp.int32), ends.astype(jnp.int32)


def kernel(x, idxs, W, b):
    idxs = idxs.astype(jnp.int32)
    counts = jnp.bincount(idxs, length=N_EXPERTS).astype(jnp.int32)
    offs = (jnp.cumsum(counts) - counts).astype(jnp.int32)
    order = jnp.argsort(idxs)
    x_sorted = jnp.take(x, order, axis=0)
    blocks, experts, lo, hi = _schedule(offs)
    y_sorted = _grouped_matmul(x_sorted, W, b, blocks, experts, lo, hi)
    inv = jnp.zeros((N_TOKENS,), jnp.int32).at[order].set(
        jnp.arange(N_TOKENS, dtype=jnp.int32))
    return jnp.take(y_sorted, inv, axis=0)


# speedup vs baseline: 3.8779x; 1.1104x over previous
import jax
import jax.numpy as jnp
from jax import lax
from jax.experimental import pallas as pl
from jax.experimental.pallas import tpu as pltpu
from jax.experimental.pallas import tpu_sc as plsc

N_TOKENS = 2048
D_IN = 768
D_OUT = 768
N_EXPERTS = 64

BLK = 128
NB = N_TOKENS // BLK
NSTEPS = NB + N_EXPERTS - 1

NC, NS = 2, 16
NW = NC * NS
TPW = N_TOKENS // NW
EC = N_EXPERTS // 16
TC_ = TPW // 16

def _sc_mesh():
    return plsc.VectorSubcoreMesh(
        core_axis_name="c", subcore_axis_name="s",
        num_cores=NC, num_subcores=NS)


def _wid():
    return lax.axis_index("s") * NC + lax.axis_index("c")


def _route_a_body(idx_hbm, hists_hbm, idx_v, hist_v):
    wid = _wid()
    base = wid * TPW
    pltpu.sync_copy(idx_hbm.at[pl.ds(base, TPW)], idx_v)
    zeros16 = jnp.zeros((16,), jnp.int32)
    ones16 = jnp.ones((16,), jnp.int32)
    for k in range(EC):
        hist_v[pl.ds(k * 16, 16)] = zeros16
    for k in range(TC_):
        v = idx_v[pl.ds(k * 16, 16)]
        plsc.addupdate_scatter(hist_v, [v], ones16)
    pltpu.sync_copy(hist_v, hists_hbm.at[wid])


def _sc_route_a(idxs):
    return pl.kernel(
        _route_a_body,
        out_type=jax.ShapeDtypeStruct((NW, N_EXPERTS), jnp.int32),
        mesh=_sc_mesh(),
        compiler_params=pltpu.CompilerParams(needs_layout_passes=False),
        scratch_types=[
            pltpu.VMEM((TPW,), jnp.int32),
            pltpu.VMEM((N_EXPERTS,), jnp.int32),
        ],
    )(idxs)


def _route_b_body(x_hbm, idx_hbm, hists_hbm,
                  xs_hbm, pos_hbm, offs_hbm,
                  idx_v, allhist_v, cnt_v, offs_v, pos_v, rows_v, tmp_v, sem):
    wid = _wid()
    base = wid * TPW
    pltpu.sync_copy(idx_hbm.at[pl.ds(base, TPW)], idx_v)
    pltpu.sync_copy(hists_hbm, allhist_v)

    zeros16 = jnp.zeros((16,), jnp.int32)
    ones16 = jnp.ones((16,), jnp.int32)
    carry = jnp.int32(0)
    for k in range(EC):
        tot = zeros16
        pre = zeros16
        for t in range(NW):
            row = allhist_v[t, pl.ds(k * 16, 16)]
            flag = (jnp.int32(t) < wid).astype(jnp.int32)
            pre = pre + row * flag
            tot = tot + row
        csum = plsc.cumsum(tot)
        offs_k = (csum - tot) + carry
        carry = carry + jnp.sum(tot)
        offs_v[pl.ds(k * 16, 16)] = offs_k
        cnt_v[pl.ds(k * 16, 16)] = offs_k + pre

    @pl.when(wid == 0)
    def _():
        pltpu.sync_copy(offs_v, offs_hbm)

    lane = lax.broadcasted_iota(jnp.int32, (16,), 0)
    for k in range(TC_):
        v = idx_v[pl.ds(k * 16, 16)]
        tmp_v[...] = v
        rank = zeros16
        for sft in range(1, 16):
            sh = plsc.load_gather(tmp_v, [(lane - sft) & 15])
            rank = rank + ((lane >= sft) & (sh == v)).astype(jnp.int32)
        basev = plsc.load_gather(cnt_v, [v])
        pos_v[pl.ds(k * 16, 16)] = basev + rank
        plsc.addupdate_scatter(cnt_v, [v], ones16)

    pltpu.sync_copy(pos_v, pos_hbm.at[pl.ds(base, TPW)])
    pltpu.sync_copy(x_hbm.at[pl.ds(base, TPW)], rows_v)
    pltpu.async_copy(rows_v, xs_hbm.at[pos_v], sem).wait()


def _sc_route_b(x, idxs, hists):
    return pl.kernel(
        _route_b_body,
        out_type=(
            jax.ShapeDtypeStruct((N_TOKENS, D_IN), jnp.float32),
            jax.ShapeDtypeStruct((N_TOKENS,), jnp.int32),
            jax.ShapeDtypeStruct((N_EXPERTS,), jnp.int32),
        ),
        mesh=_sc_mesh(),
        compiler_params=pltpu.CompilerParams(needs_layout_passes=False),
        scratch_types=[
            pltpu.VMEM((TPW,), jnp.int32),
            pltpu.VMEM((NW, N_EXPERTS), jnp.int32),
            pltpu.VMEM((N_EXPERTS,), jnp.int32),
            pltpu.VMEM((N_EXPERTS,), jnp.int32),
            pltpu.VMEM((TPW,), jnp.int32),
            pltpu.VMEM((TPW, D_IN), jnp.float32),
            pltpu.VMEM((16,), jnp.int32),
            pltpu.SemaphoreType.DMA,
        ],
    )(x, idxs, hists)


def _unsort_body(ys_hbm, pos_hbm, out_hbm, pos_v, rows_v, sem):
    wid = _wid()
    base = wid * TPW
    pltpu.sync_copy(pos_hbm.at[pl.ds(base, TPW)], pos_v)
    pltpu.async_copy(ys_hbm.at[pos_v], rows_v, sem).wait()
    pltpu.sync_copy(rows_v, out_hbm.at[pl.ds(base, TPW)])


def _sc_unsort(y_sorted, pos):
    return pl.kernel(
        _unsort_body,
        out_type=jax.ShapeDtypeStruct((N_TOKENS, D_OUT), jnp.float32),
        mesh=_sc_mesh(),
        compiler_params=pltpu.CompilerParams(needs_layout_passes=False),
        scratch_types=[
            pltpu.VMEM((TPW,), jnp.int32),
            pltpu.VMEM((TPW, D_OUT), jnp.float32),
            pltpu.SemaphoreType.DMA,
        ],
    )(y_sorted, pos)


def _gmm_body(blk_ref, exp_ref, lo_ref, hi_ref, xs_ref, w_ref, b_ref, o_ref):
    s = pl.program_id(0)
    lo = lo_ref[s]
    hi = hi_ref[s]
    blk = blk_ref[s]

    @pl.when(hi > lo)
    def _():
        rows = blk * BLK + lax.broadcasted_iota(jnp.int32, (BLK, 1), 0)
        mask = (rows >= lo) & (rows < hi)
        y = lax.dot_general(
            xs_ref[...], w_ref[0],
            (((1,), (1,)), ((), ())),
            preferred_element_type=jnp.float32,
        )
        y = jnp.maximum(y + b_ref[0], 0.0)
        o_ref[...] = jnp.where(mask, y, o_ref[...])


def _grouped_matmul(x_sorted, W, b, blocks, experts, lo, hi):
    grid_spec = pltpu.PrefetchScalarGridSpec(
        num_scalar_prefetch=4,
        grid=(NSTEPS,),
        in_specs=[
            pl.BlockSpec((BLK, D_IN), lambda s, bs, es, ls, hs: (bs[s], 0)),
            pl.BlockSpec((1, D_OUT, D_IN), lambda s, bs, es, ls, hs: (es[s], 0, 0)),
            pl.BlockSpec((1, 1, D_OUT), lambda s, bs, es, ls, hs: (es[s], 0, 0)),
        ],
        out_specs=pl.BlockSpec((BLK, D_OUT), lambda s, bs, es, ls, hs: (bs[s], 0)),
    )
    return pl.pallas_call(
        _gmm_body,
        grid_spec=grid_spec,
        out_shape=jax.ShapeDtypeStruct((N_TOKENS, D_OUT), jnp.float32),
        compiler_params=pltpu.CompilerParams(
            dimension_semantics=("arbitrary",),
        ),
    )(blocks, experts, lo, hi, x_sorted, W, b.reshape(N_EXPERTS, 1, D_OUT))


def _schedule(offs):
    bounds = jnp.arange(1, NB, dtype=jnp.int32) * BLK
    starts = jnp.sort(jnp.concatenate([offs, bounds]))
    ends = jnp.concatenate([starts[1:], jnp.array([N_TOKENS], jnp.int32)])
    experts = jnp.clip(
        jnp.searchsorted(offs, starts, side="right").astype(jnp.int32) - 1,
        0, N_EXPERTS - 1)
    blocks = jnp.clip(starts // BLK, 0, NB - 1).astype(jnp.int32)
    return blocks, experts, starts.astype(jnp.int32), ends.astype(jnp.int32)


def kernel(x, idxs, W, b):
    idxs = idxs.astype(jnp.int32)
    hists = _sc_route_a(idxs)
    x_sorted, pos, offs = _sc_route_b(x, idxs, hists)
    blocks, experts, lo, hi = _schedule(offs)
    y_sorted = _grouped_matmul(x_sorted, W, b, blocks, experts, lo, hi)
    return _sc_unsort(y_sorted, pos)


# speedup vs baseline: 3.9027x; 1.0064x over previous
import jax
import jax.numpy as jnp
from jax import lax
from jax.experimental import pallas as pl
from jax.experimental.pallas import tpu as pltpu
from jax.experimental.pallas import tpu_sc as plsc

N_TOKENS = 2048
D_IN = 768
D_OUT = 768
N_EXPERTS = 64

BLK = 128
NB = N_TOKENS // BLK
NSTEPS = NB + N_EXPERTS - 1

NC, NS = 2, 16
NW = NC * NS
TPW = N_TOKENS // NW
EC = N_EXPERTS // 16
TC_ = TPW // 16

def _sc_mesh():
    return plsc.VectorSubcoreMesh(
        core_axis_name="c", subcore_axis_name="s",
        num_cores=NC, num_subcores=NS)


def _wid():
    return lax.axis_index("s") * NC + lax.axis_index("c")


def _route_a_body(idx_hbm, hists_hbm, idx_v, hist_v):
    wid = _wid()
    base = wid * TPW
    pltpu.sync_copy(idx_hbm.at[pl.ds(base, TPW)], idx_v)
    zeros16 = jnp.zeros((16,), jnp.int32)
    ones16 = jnp.ones((16,), jnp.int32)
    for k in range(EC):
        hist_v[pl.ds(k * 16, 16)] = zeros16
    for k in range(TC_):
        v = idx_v[pl.ds(k * 16, 16)]
        plsc.addupdate_scatter(hist_v, [v], ones16)
    pltpu.sync_copy(hist_v, hists_hbm.at[wid])


def _sc_route_a(idxs):
    return pl.kernel(
        _route_a_body,
        out_type=jax.ShapeDtypeStruct((NW, N_EXPERTS), jnp.int32),
        mesh=_sc_mesh(),
        compiler_params=pltpu.CompilerParams(needs_layout_passes=False),
        scratch_types=[
            pltpu.VMEM((TPW,), jnp.int32),
            pltpu.VMEM((N_EXPERTS,), jnp.int32),
        ],
    )(idxs)


def _route_b_body(x_hbm, idx_hbm, hists_hbm,
                  xs_hbm, pos_hbm, offs_hbm,
                  idx_v, allhist_v, cnt_v, offs_v, pos_v, rows_v, tmp_v, sem):
    wid = _wid()
    base = wid * TPW
    pltpu.sync_copy(idx_hbm.at[pl.ds(base, TPW)], idx_v)
    pltpu.sync_copy(hists_hbm, allhist_v)

    zeros16 = jnp.zeros((16,), jnp.int32)
    ones16 = jnp.ones((16,), jnp.int32)
    carry = jnp.int32(0)
    for k in range(EC):
        tot = zeros16
        pre = zeros16
        for t in range(NW):
            row = allhist_v[t, pl.ds(k * 16, 16)]
            flag = (jnp.int32(t) < wid).astype(jnp.int32)
            pre = pre + row * flag
            tot = tot + row
        csum = plsc.cumsum(tot)
        offs_k = (csum - tot) + carry
        carry = carry + jnp.sum(tot)
        offs_v[pl.ds(k * 16, 16)] = offs_k
        cnt_v[pl.ds(k * 16, 16)] = offs_k + pre

    @pl.when(wid == 0)
    def _():
        pltpu.sync_copy(offs_v, offs_hbm)

    lane = lax.broadcasted_iota(jnp.int32, (16,), 0)
    for k in range(TC_):
        v = idx_v[pl.ds(k * 16, 16)]
        tmp_v[...] = v
        rank = zeros16
        for sft in range(1, 16):
            sh = plsc.load_gather(tmp_v, [(lane - sft) & 15])
            rank = rank + ((lane >= sft) & (sh == v)).astype(jnp.int32)
        basev = plsc.load_gather(cnt_v, [v])
        pos_v[pl.ds(k * 16, 16)] = basev + rank
        plsc.addupdate_scatter(cnt_v, [v], ones16)

    pltpu.sync_copy(pos_v, pos_hbm.at[pl.ds(base, TPW)])
    pltpu.sync_copy(x_hbm.at[pl.ds(base, TPW)], rows_v)
    pltpu.async_copy(rows_v, xs_hbm.at[pos_v], sem).wait()


def _sc_route_b(x, idxs, hists):
    return pl.kernel(
        _route_b_body,
        out_type=(
            jax.ShapeDtypeStruct((N_TOKENS, D_IN), jnp.float32),
            jax.ShapeDtypeStruct((N_TOKENS,), jnp.int32),
            jax.ShapeDtypeStruct((N_EXPERTS,), jnp.int32),
        ),
        mesh=_sc_mesh(),
        compiler_params=pltpu.CompilerParams(needs_layout_passes=False),
        scratch_types=[
            pltpu.VMEM((TPW,), jnp.int32),
            pltpu.VMEM((NW, N_EXPERTS), jnp.int32),
            pltpu.VMEM((N_EXPERTS,), jnp.int32),
            pltpu.VMEM((N_EXPERTS,), jnp.int32),
            pltpu.VMEM((TPW,), jnp.int32),
            pltpu.VMEM((TPW, D_IN), jnp.float32),
            pltpu.VMEM((16,), jnp.int32),
            pltpu.SemaphoreType.DMA,
        ],
    )(x, idxs, hists)


def _unsort_body(ys_hbm, pos_hbm, out_hbm, pos_v, rows_v, sem):
    wid = _wid()
    base = wid * TPW
    pltpu.sync_copy(pos_hbm.at[pl.ds(base, TPW)], pos_v)
    pltpu.async_copy(ys_hbm.at[pos_v], rows_v, sem).wait()
    pltpu.sync_copy(rows_v, out_hbm.at[pl.ds(base, TPW)])


def _sc_unsort(y_sorted, pos):
    return pl.kernel(
        _unsort_body,
        out_type=jax.ShapeDtypeStruct((N_TOKENS, D_OUT), jnp.float32),
        mesh=_sc_mesh(),
        compiler_params=pltpu.CompilerParams(needs_layout_passes=False),
        scratch_types=[
            pltpu.VMEM((TPW,), jnp.int32),
            pltpu.VMEM((TPW, D_OUT), jnp.float32),
            pltpu.SemaphoreType.DMA,
        ],
    )(y_sorted, pos)


NWS = 2
WS = D_OUT // NWS


def _gmm_body(blk_ref, exp_ref, lo_ref, hi_ref, xs_ref, *rest):
    *w_refs, b_ref, o_ref = rest
    s = pl.program_id(0)
    lo = lo_ref[s]
    hi = hi_ref[s]
    blk = blk_ref[s]

    @pl.when(hi > lo)
    def _():
        rows = blk * BLK + lax.broadcasted_iota(jnp.int32, (BLK, 1), 0)
        mask = (rows >= lo) & (rows < hi)
        x_blk = xs_ref[...]
        y = jnp.concatenate(
            [
                lax.dot_general(
                    x_blk, w_ref[0],
                    (((1,), (1,)), ((), ())),
                    preferred_element_type=jnp.float32,
                )
                for w_ref in w_refs
            ],
            axis=1,
        )
        y = jnp.maximum(y + b_ref[0], 0.0)
        o_ref[...] = jnp.where(mask, y, o_ref[...])


def _grouped_matmul(x_sorted, W, b, blocks, experts, lo, hi):
    grid_spec = pltpu.PrefetchScalarGridSpec(
        num_scalar_prefetch=4,
        grid=(NSTEPS,),
        in_specs=[
            pl.BlockSpec((BLK, D_IN), lambda s, bs, es, ls, hs: (bs[s], 0)),
        ] + [
            pl.BlockSpec((1, WS, D_IN),
                         lambda s, bs, es, ls, hs, i=i: (es[s], i, 0))
            for i in range(NWS)
        ] + [
            pl.BlockSpec((1, 1, D_OUT), lambda s, bs, es, ls, hs: (es[s], 0, 0)),
        ],
        out_specs=pl.BlockSpec((BLK, D_OUT), lambda s, bs, es, ls, hs: (bs[s], 0)),
    )
    w_parts = [W] * NWS
    return pl.pallas_call(
        _gmm_body,
        grid_spec=grid_spec,
        out_shape=jax.ShapeDtypeStruct((N_TOKENS, D_OUT), jnp.float32),
        compiler_params=pltpu.CompilerParams(
            dimension_semantics=("arbitrary",),
        ),
    )(blocks, experts, lo, hi, x_sorted, *w_parts,
      b.reshape(N_EXPERTS, 1, D_OUT))


def _schedule(offs):
    bounds = jnp.arange(1, NB, dtype=jnp.int32) * BLK
    starts = jnp.sort(jnp.concatenate([offs, bounds]))
    ends = jnp.concatenate([starts[1:], jnp.array([N_TOKENS], jnp.int32)])
    experts = jnp.clip(
        jnp.searchsorted(offs, starts, side="right").astype(jnp.int32) - 1,
        0, N_EXPERTS - 1)
    blocks = jnp.clip(starts // BLK, 0, NB - 1).astype(jnp.int32)
    return blocks, experts, starts.astype(jnp.int32), ends.astype(jnp.int32)


def kernel(x, idxs, W, b):
    idxs = idxs.astype(jnp.int32)
    hists = _sc_route_a(idxs)
    x_sorted, pos, offs = _sc_route_b(x, idxs, hists)
    blocks, experts, lo, hi = _schedule(offs)
    y_sorted = _grouped_matmul(x_sorted, W, b, blocks, experts, lo, hi)
    return _sc_unsort(y_sorted, pos)


# speedup vs baseline: 3.9603x; 1.0147x over previous
import jax
import jax.numpy as jnp
from jax import lax
from jax.experimental import pallas as pl
from jax.experimental.pallas import tpu as pltpu
from jax.experimental.pallas import tpu_sc as plsc

N_TOKENS = 2048
D_IN = 768
D_OUT = 768
N_EXPERTS = 64

BLK = 256
NB = N_TOKENS // BLK
NSTEPS = NB + N_EXPERTS - 1

NC, NS = 2, 16
NW = NC * NS
TPW = N_TOKENS // NW
EC = N_EXPERTS // 16
TC_ = TPW // 16

def _sc_mesh():
    return plsc.VectorSubcoreMesh(
        core_axis_name="c", subcore_axis_name="s",
        num_cores=NC, num_subcores=NS)


def _wid():
    return lax.axis_index("s") * NC + lax.axis_index("c")


def _route_a_body(idx_hbm, hists_hbm, idx_v, hist_v):
    wid = _wid()
    base = wid * TPW
    pltpu.sync_copy(idx_hbm.at[pl.ds(base, TPW)], idx_v)
    zeros16 = jnp.zeros((16,), jnp.int32)
    ones16 = jnp.ones((16,), jnp.int32)
    for k in range(EC):
        hist_v[pl.ds(k * 16, 16)] = zeros16
    for k in range(TC_):
        v = idx_v[pl.ds(k * 16, 16)]
        plsc.addupdate_scatter(hist_v, [v], ones16)
    pltpu.sync_copy(hist_v, hists_hbm.at[wid])


def _sc_route_a(idxs):
    return pl.kernel(
        _route_a_body,
        out_type=jax.ShapeDtypeStruct((NW, N_EXPERTS), jnp.int32),
        mesh=_sc_mesh(),
        compiler_params=pltpu.CompilerParams(needs_layout_passes=False),
        scratch_types=[
            pltpu.VMEM((TPW,), jnp.int32),
            pltpu.VMEM((N_EXPERTS,), jnp.int32),
        ],
    )(idxs)


def _route_b_body(x_hbm, idx_hbm, hists_hbm,
                  xs_hbm, pos_hbm, offs_hbm,
                  idx_v, allhist_v, cnt_v, offs_v, pos_v, rows_v, tmp_v, sem):
    wid = _wid()
    base = wid * TPW
    pltpu.sync_copy(idx_hbm.at[pl.ds(base, TPW)], idx_v)
    pltpu.sync_copy(hists_hbm, allhist_v)

    zeros16 = jnp.zeros((16,), jnp.int32)
    ones16 = jnp.ones((16,), jnp.int32)
    carry = jnp.int32(0)
    for k in range(EC):
        tot = zeros16
        pre = zeros16
        for t in range(NW):
            row = allhist_v[t, pl.ds(k * 16, 16)]
            flag = (jnp.int32(t) < wid).astype(jnp.int32)
            pre = pre + row * flag
            tot = tot + row
        csum = plsc.cumsum(tot)
        offs_k = (csum - tot) + carry
        carry = carry + jnp.sum(tot)
        offs_v[pl.ds(k * 16, 16)] = offs_k
        cnt_v[pl.ds(k * 16, 16)] = offs_k + pre

    @pl.when(wid == 0)
    def _():
        pltpu.sync_copy(offs_v, offs_hbm)

    lane = lax.broadcasted_iota(jnp.int32, (16,), 0)
    for k in range(TC_):
        v = idx_v[pl.ds(k * 16, 16)]
        tmp_v[...] = v
        rank = zeros16
        for sft in range(1, 16):
            sh = plsc.load_gather(tmp_v, [(lane - sft) & 15])
            rank = rank + ((lane >= sft) & (sh == v)).astype(jnp.int32)
        basev = plsc.load_gather(cnt_v, [v])
        pos_v[pl.ds(k * 16, 16)] = basev + rank
        plsc.addupdate_scatter(cnt_v, [v], ones16)

    pltpu.sync_copy(pos_v, pos_hbm.at[pl.ds(base, TPW)])
    pltpu.sync_copy(x_hbm.at[pl.ds(base, TPW)], rows_v)
    pltpu.async_copy(rows_v, xs_hbm.at[pos_v], sem).wait()


def _sc_route_b(x, idxs, hists):
    return pl.kernel(
        _route_b_body,
        out_type=(
            jax.ShapeDtypeStruct((N_TOKENS, D_IN), jnp.float32),
            jax.ShapeDtypeStruct((N_TOKENS,), jnp.int32),
            jax.ShapeDtypeStruct((N_EXPERTS,), jnp.int32),
        ),
        mesh=_sc_mesh(),
        compiler_params=pltpu.CompilerParams(needs_layout_passes=False),
        scratch_types=[
            pltpu.VMEM((TPW,), jnp.int32),
            pltpu.VMEM((NW, N_EXPERTS), jnp.int32),
            pltpu.VMEM((N_EXPERTS,), jnp.int32),
            pltpu.VMEM((N_EXPERTS,), jnp.int32),
            pltpu.VMEM((TPW,), jnp.int32),
            pltpu.VMEM((TPW, D_IN), jnp.float32),
            pltpu.VMEM((16,), jnp.int32),
            pltpu.SemaphoreType.DMA,
        ],
    )(x, idxs, hists)


def _unsort_body(ys_hbm, pos_hbm, out_hbm, pos_v, rows_v, sem):
    wid = _wid()
    base = wid * TPW
    pltpu.sync_copy(pos_hbm.at[pl.ds(base, TPW)], pos_v)
    pltpu.async_copy(ys_hbm.at[pos_v], rows_v, sem).wait()
    pltpu.sync_copy(rows_v, out_hbm.at[pl.ds(base, TPW)])


def _sc_unsort(y_sorted, pos):
    return pl.kernel(
        _unsort_body,
        out_type=jax.ShapeDtypeStruct((N_TOKENS, D_OUT), jnp.float32),
        mesh=_sc_mesh(),
        compiler_params=pltpu.CompilerParams(needs_layout_passes=False),
        scratch_types=[
            pltpu.VMEM((TPW,), jnp.int32),
            pltpu.VMEM((TPW, D_OUT), jnp.float32),
            pltpu.SemaphoreType.DMA,
        ],
    )(y_sorted, pos)


NWS = 2
WS = D_OUT // NWS


def _gmm_body(blk_ref, exp_ref, lo_ref, hi_ref, xs_ref, *rest):
    *w_refs, b_ref, o_ref = rest
    s = pl.program_id(0)
    lo = lo_ref[s]
    hi = hi_ref[s]
    blk = blk_ref[s]

    @pl.when(hi > lo)
    def _():
        rows = blk * BLK + lax.broadcasted_iota(jnp.int32, (BLK, 1), 0)
        mask = (rows >= lo) & (rows < hi)
        x_blk = xs_ref[...]
        y = jnp.concatenate(
            [
                lax.dot_general(
                    x_blk, w_ref[0],
                    (((1,), (1,)), ((), ())),
                    preferred_element_type=jnp.float32,
                )
                for w_ref in w_refs
            ],
            axis=1,
        )
        y = jnp.maximum(y + b_ref[0], 0.0)
        o_ref[...] = jnp.where(mask, y, o_ref[...])


def _grouped_matmul(x_sorted, W, b, blocks, experts, lo, hi):
    grid_spec = pltpu.PrefetchScalarGridSpec(
        num_scalar_prefetch=4,
        grid=(NSTEPS,),
        in_specs=[
            pl.BlockSpec((BLK, D_IN), lambda s, bs, es, ls, hs: (bs[s], 0)),
        ] + [
            pl.BlockSpec((1, WS, D_IN),
                         lambda s, bs, es, ls, hs, i=i: (es[s], i, 0))
            for i in range(NWS)
        ] + [
            pl.BlockSpec((1, 1, D_OUT), lambda s, bs, es, ls, hs: (es[s], 0, 0)),
        ],
        out_specs=pl.BlockSpec((BLK, D_OUT), lambda s, bs, es, ls, hs: (bs[s], 0)),
    )
    w_parts = [W] * NWS
    return pl.pallas_call(
        _gmm_body,
        grid_spec=grid_spec,
        out_shape=jax.ShapeDtypeStruct((N_TOKENS, D_OUT), jnp.float32),
        compiler_params=pltpu.CompilerParams(
            dimension_semantics=("arbitrary",),
        ),
    )(blocks, experts, lo, hi, x_sorted, *w_parts,
      b.reshape(N_EXPERTS, 1, D_OUT))


def _schedule(offs):
    bounds = jnp.arange(1, NB, dtype=jnp.int32) * BLK
    starts = jnp.sort(jnp.concatenate([offs, bounds]))
    ends = jnp.concatenate([starts[1:], jnp.array([N_TOKENS], jnp.int32)])
    experts = jnp.clip(
        jnp.searchsorted(offs, starts, side="right").astype(jnp.int32) - 1,
        0, N_EXPERTS - 1)
    blocks = jnp.clip(starts // BLK, 0, NB - 1).astype(jnp.int32)
    return blocks, experts, starts.astype(jnp.int32), ends.astype(jnp.int32)


def kernel(x, idxs, W, b):
    idxs = idxs.astype(jnp.int32)
    hists = _sc_route_a(idxs)
    x_sorted, pos, offs = _sc_route_b(x, idxs, hists)
    blocks, experts, lo, hi = _schedule(offs)
    y_sorted = _grouped_matmul(x_sorted, W, b, blocks, experts, lo, hi)
    return _sc_unsort(y_sorted, pos)


# speedup vs baseline: 4.4149x; 1.1148x over previous
import jax
import jax.numpy as jnp
from jax import lax
from jax.experimental import pallas as pl
from jax.experimental.pallas import tpu as pltpu
from jax.experimental.pallas import tpu_sc as plsc

N_TOKENS = 2048
D_IN = 768
D_OUT = 768
N_EXPERTS = 64

BLK = 256
NB = N_TOKENS // BLK
NSTEPS = NB + N_EXPERTS - 1

NC, NS = 2, 16
NW = NC * NS
TPW = N_TOKENS // NW
EC = N_EXPERTS // 16
TC_ = TPW // 16

def _sc_mesh():
    return plsc.VectorSubcoreMesh(
        core_axis_name="c", subcore_axis_name="s",
        num_cores=NC, num_subcores=NS)


def _wid():
    return lax.axis_index("s") * NC + lax.axis_index("c")


def _route_a_body(idx_hbm, hists_hbm, idx_v, hist_v):
    wid = _wid()
    base = wid * TPW
    pltpu.sync_copy(idx_hbm.at[pl.ds(base, TPW)], idx_v)
    zeros16 = jnp.zeros((16,), jnp.int32)
    ones16 = jnp.ones((16,), jnp.int32)
    for k in range(EC):
        hist_v[pl.ds(k * 16, 16)] = zeros16
    for k in range(TC_):
        v = idx_v[pl.ds(k * 16, 16)]
        plsc.addupdate_scatter(hist_v, [v], ones16)
    pltpu.sync_copy(hist_v, hists_hbm.at[wid])


def _sc_route_a(idxs):
    return pl.kernel(
        _route_a_body,
        out_type=jax.ShapeDtypeStruct((NW, N_EXPERTS), jnp.int32),
        mesh=_sc_mesh(),
        compiler_params=pltpu.CompilerParams(needs_layout_passes=False),
        scratch_types=[
            pltpu.VMEM((TPW,), jnp.int32),
            pltpu.VMEM((N_EXPERTS,), jnp.int32),
        ],
    )(idxs)


def _route_b_body(x_hbm, idx_hbm, hists_hbm,
                  xs_hbm, pos_hbm,
                  idx_v, allhist_v, cnt_v, pos_v, rows_v, tmp_v, sem):
    wid = _wid()
    base = wid * TPW
    pltpu.sync_copy(idx_hbm.at[pl.ds(base, TPW)], idx_v)
    pltpu.sync_copy(hists_hbm, allhist_v)

    zeros16 = jnp.zeros((16,), jnp.int32)
    ones16 = jnp.ones((16,), jnp.int32)
    carry = jnp.int32(0)
    for k in range(EC):
        tot = zeros16
        pre = zeros16
        for t in range(NW):
            row = allhist_v[t, pl.ds(k * 16, 16)]
            flag = (jnp.int32(t) < wid).astype(jnp.int32)
            pre = pre + row * flag
            tot = tot + row
        csum = plsc.cumsum(tot)
        offs_k = (csum - tot) + carry
        carry = carry + jnp.sum(tot)
        cnt_v[pl.ds(k * 16, 16)] = offs_k + pre

    lane = lax.broadcasted_iota(jnp.int32, (16,), 0)
    for k in range(TC_):
        v = idx_v[pl.ds(k * 16, 16)]
        tmp_v[...] = v
        rank = zeros16
        for sft in range(1, 16):
            sh = plsc.load_gather(tmp_v, [(lane - sft) & 15])
            rank = rank + ((lane >= sft) & (sh == v)).astype(jnp.int32)
        basev = plsc.load_gather(cnt_v, [v])
        pos_v[pl.ds(k * 16, 16)] = basev + rank
        plsc.addupdate_scatter(cnt_v, [v], ones16)

    pltpu.sync_copy(pos_v, pos_hbm.at[pl.ds(base, TPW)])
    pltpu.sync_copy(x_hbm.at[pl.ds(base, TPW)], rows_v)
    pltpu.async_copy(rows_v, xs_hbm.at[pos_v], sem).wait()


def _sc_route_b(x, idxs, hists):
    return pl.kernel(
        _route_b_body,
        out_type=(
            jax.ShapeDtypeStruct((N_TOKENS, D_IN), jnp.float32),
            jax.ShapeDtypeStruct((N_TOKENS,), jnp.int32),
        ),
        mesh=_sc_mesh(),
        compiler_params=pltpu.CompilerParams(needs_layout_passes=False),
        scratch_types=[
            pltpu.VMEM((TPW,), jnp.int32),
            pltpu.VMEM((NW, N_EXPERTS), jnp.int32),
            pltpu.VMEM((N_EXPERTS,), jnp.int32),
            pltpu.VMEM((TPW,), jnp.int32),
            pltpu.VMEM((TPW, D_IN), jnp.float32),
            pltpu.VMEM((16,), jnp.int32),
            pltpu.SemaphoreType.DMA,
        ],
    )(x, idxs, hists)


def _unsort_body(ys_hbm, pos_hbm, out_hbm, pos_v, rows_v, sem):
    wid = _wid()
    base = wid * TPW
    pltpu.sync_copy(pos_hbm.at[pl.ds(base, TPW)], pos_v)
    pltpu.async_copy(ys_hbm.at[pos_v], rows_v, sem).wait()
    pltpu.sync_copy(rows_v, out_hbm.at[pl.ds(base, TPW)])


def _sc_unsort(y_sorted, pos):
    return pl.kernel(
        _unsort_body,
        out_type=jax.ShapeDtypeStruct((N_TOKENS, D_OUT), jnp.float32),
        mesh=_sc_mesh(),
        compiler_params=pltpu.CompilerParams(needs_layout_passes=False),
        scratch_types=[
            pltpu.VMEM((TPW,), jnp.int32),
            pltpu.VMEM((TPW, D_OUT), jnp.float32),
            pltpu.SemaphoreType.DMA,
        ],
    )(y_sorted, pos)


NWS = 2
WS = D_OUT // NWS


def _gmm_body(blk_ref, exp_ref, lo_ref, hi_ref, xs_ref, *rest):
    *w_refs, b_ref, o_ref = rest
    s = pl.program_id(0)
    lo = lo_ref[s]
    hi = hi_ref[s]
    blk = blk_ref[s]

    @pl.when(hi > lo)
    def _():
        rows = blk * BLK + lax.broadcasted_iota(jnp.int32, (BLK, 1), 0)
        mask = (rows >= lo) & (rows < hi)
        x_blk = xs_ref[...]
        y = jnp.concatenate(
            [
                lax.dot_general(
                    x_blk, w_ref[0],
                    (((1,), (1,)), ((), ())),
                    preferred_element_type=jnp.float32,
                )
                for w_ref in w_refs
            ],
            axis=1,
        )
        y = jnp.maximum(y + b_ref[0], 0.0)
        o_ref[...] = jnp.where(mask, y, o_ref[...])


def _grouped_matmul(x_sorted, W, b, blocks, experts, lo, hi):
    grid_spec = pltpu.PrefetchScalarGridSpec(
        num_scalar_prefetch=4,
        grid=(NSTEPS,),
        in_specs=[
            pl.BlockSpec((BLK, D_IN), lambda s, bs, es, ls, hs: (bs[s], 0)),
        ] + [
            pl.BlockSpec((1, WS, D_IN),
                         lambda s, bs, es, ls, hs, i=i: (es[s], i, 0))
            for i in range(NWS)
        ] + [
            pl.BlockSpec((1, 1, D_OUT), lambda s, bs, es, ls, hs: (es[s], 0, 0)),
        ],
        out_specs=pl.BlockSpec((BLK, D_OUT), lambda s, bs, es, ls, hs: (bs[s], 0)),
    )
    w_parts = [W] * NWS
    return pl.pallas_call(
        _gmm_body,
        grid_spec=grid_spec,
        out_shape=jax.ShapeDtypeStruct((N_TOKENS, D_OUT), jnp.float32),
        compiler_params=pltpu.CompilerParams(
            dimension_semantics=("arbitrary",),
        ),
    )(blocks, experts, lo, hi, x_sorted, *w_parts,
      b.reshape(N_EXPERTS, 1, D_OUT))


def _schedule(hists):
    tot = jnp.sum(hists, axis=0)
    offs = (jnp.cumsum(tot) - tot).astype(jnp.int32)
    keys_e = offs * 128 + jnp.arange(N_EXPERTS, dtype=jnp.int32)
    bounds = jnp.arange(1, NB, dtype=jnp.int32) * BLK
    keys_b = bounds * 128 + 64
    keys = jnp.sort(jnp.concatenate([keys_e, keys_b]))
    starts = keys >> 7
    is_exp = (keys & 64) == 0
    eid = jnp.where(is_exp, keys & 63, -1)
    experts = jax.lax.cummax(eid)
    blocks = jnp.clip(starts // BLK, 0, NB - 1)
    ends = jnp.concatenate([starts[1:], jnp.array([N_TOKENS], jnp.int32)])
    return blocks, experts, starts, ends


def kernel(x, idxs, W, b):
    idxs = idxs.astype(jnp.int32)
    hists = _sc_route_a(idxs)
    x_sorted, pos = _sc_route_b(x, idxs, hists)
    blocks, experts, lo, hi = _schedule(hists)
    y_sorted = _grouped_matmul(x_sorted, W, b, blocks, experts, lo, hi)
    return _sc_unsort(y_sorted, pos)
